# baseline (speedup 1.0000x reference)
import functools

import jax
import jax.numpy as jnp
import numpy as np
from jax import lax
from jax.experimental import pallas as pl
from jax.experimental.pallas import tpu as pltpu
from jax.experimental.pallas import tpu_sc as plsc

_NROW, _NCOL = 8192, 2048
_FRACTION = 0.2
_N = int(_NCOL * _FRACTION)
_NIDX = 432

_NC, _NS = 2, 16
_NW = _NC * _NS
_ROWS_PER_W = _NROW // _NW
_B = 16
_NBATCH = _ROWS_PER_W // _B


def _threefry2x32(k0, k1, x0, x1):
    def rol(x, d):
        return ((x << np.uint32(d)) | (x >> np.uint32(32 - d))).astype(np.uint32)

    ks0, ks1 = np.uint32(k0), np.uint32(k1)
    ks2 = np.uint32(ks0 ^ ks1 ^ np.uint32(0x1BD11BDA))
    x0 = (x0 + ks0).astype(np.uint32)
    x1 = (x1 + ks1).astype(np.uint32)
    rots = ((13, 15, 26, 6), (17, 29, 16, 24))
    inject = ((ks1, ks2), (ks2, ks0), (ks0, ks1), (ks1, ks2), (ks2, ks0))
    for i in range(5):
        for d in rots[i % 2]:
            x0 = (x0 + x1).astype(np.uint32)
            x1 = rol(x1, d)
            x1 = (x1 ^ x0).astype(np.uint32)
        a, b = inject[i]
        x0 = (x0 + a).astype(np.uint32)
        x1 = (x1 + b + np.uint32(i + 1)).astype(np.uint32)
    return x0, x1


@functools.lru_cache(maxsize=1)
def _idx_flat() -> np.ndarray:
    counts = np.arange(_NROW * _NCOL, dtype=np.uint64)
    hi = (counts >> np.uint64(32)).astype(np.uint32)
    lo = (counts & np.uint64(0xFFFFFFFF)).astype(np.uint32)
    x0, x1 = _threefry2x32(0, 42, hi, lo)
    bits = x0 ^ x1
    fb = (bits >> np.uint32(9)) | np.uint32(0x3F800000)
    u = (fb.view(np.float32) - np.float32(1.0)).reshape(_NROW, _NCOL)
    idx = np.argsort(u, axis=1, kind="stable")[:, :_N].astype(np.int32)
    pad = np.repeat(idx[:, -1:], _NIDX - _N, axis=1)
    idx = np.concatenate([idx, pad], axis=1)
    row_local = (np.arange(_NROW, dtype=np.int32) % _B)[:, None]
    return (idx + row_local * _NCOL).reshape(-1)


def _sc_body(x_hbm, idx_hbm, out_hbm, xbuf, ibuf):
    wid = lax.axis_index("s") * _NC + lax.axis_index("c")
    xbase = wid * (_ROWS_PER_W * _NCOL)
    ibase = wid * (_ROWS_PER_W * _NIDX)
    zeros = jnp.zeros((16,), jnp.float32)

    def batch_body(b, carry):
        xoff = xbase + b * (_B * _NCOL)
        ioff = ibase + b * (_B * _NIDX)
        pltpu.sync_copy(x_hbm.at[pl.ds(xoff, _B * _NCOL)], xbuf)
        pltpu.sync_copy(idx_hbm.at[pl.ds(ioff, _B * _NIDX)], ibuf)

        def scat(j, c):
            vidx = ibuf[pl.ds(j * 16, 16)]
            plsc.store_scatter(xbuf, [vidx], zeros)
            return c

        lax.fori_loop(0, _B * _NIDX // 16, scat, 0)
        pltpu.sync_copy(xbuf, out_hbm.at[pl.ds(xoff, _B * _NCOL)])
        return carry

    lax.fori_loop(0, _NBATCH, batch_body, 0)


@functools.partial(
    pl.kernel,
    mesh=plsc.VectorSubcoreMesh(core_axis_name="c", subcore_axis_name="s"),
    out_type=jax.ShapeDtypeStruct((_NROW * _NCOL,), jnp.float32),
    scratch_types=[
        pltpu.VMEM((_B * _NCOL,), jnp.float32),
        pltpu.VMEM((_B * _NIDX,), jnp.int32),
    ],
    compiler_params=pltpu.CompilerParams(needs_layout_passes=False),
)
def _sc_mask(x_hbm, idx_hbm, out_hbm, xbuf, ibuf):
    _sc_body(x_hbm, idx_hbm, out_hbm, xbuf, ibuf)


def kernel(x):
    idx = _idx_flat()
    out = _sc_mask(x.reshape(-1), idx)
    return out.reshape(_NROW, _NCOL)


# speedup vs baseline: 1.3065x; 1.3065x over previous
import functools

import jax
import jax.numpy as jnp
import numpy as np
from jax import lax
from jax.experimental import pallas as pl
from jax.experimental.pallas import tpu as pltpu
from jax.experimental.pallas import tpu_sc as plsc

_NROW, _NCOL = 8192, 2048
_FRACTION = 0.2
_N = int(_NCOL * _FRACTION)
_NIDX = 416

_NC, _NS = 2, 16
_NW = _NC * _NS
_ROWS_PER_W = _NROW // _NW
_B = 8
_NBATCH = _ROWS_PER_W // _B
_NBUF = 4


def _threefry2x32(k0, k1, x0, x1):
    def rol(x, d):
        return ((x << np.uint32(d)) | (x >> np.uint32(32 - d))).astype(np.uint32)

    ks0, ks1 = np.uint32(k0), np.uint32(k1)
    ks2 = np.uint32(ks0 ^ ks1 ^ np.uint32(0x1BD11BDA))
    x0 = (x0 + ks0).astype(np.uint32)
    x1 = (x1 + ks1).astype(np.uint32)
    rots = ((13, 15, 26, 6), (17, 29, 16, 24))
    inject = ((ks1, ks2), (ks2, ks0), (ks0, ks1), (ks1, ks2), (ks2, ks0))
    for i in range(5):
        for d in rots[i % 2]:
            x0 = (x0 + x1).astype(np.uint32)
            x1 = rol(x1, d)
            x1 = (x1 ^ x0).astype(np.uint32)
        a, b = inject[i]
        x0 = (x0 + a).astype(np.uint32)
        x1 = (x1 + b + np.uint32(i + 1)).astype(np.uint32)
    return x0, x1


@functools.lru_cache(maxsize=1)
def _idx_flat() -> np.ndarray:
    counts = np.arange(_NROW * _NCOL, dtype=np.uint64)
    hi = (counts >> np.uint64(32)).astype(np.uint32)
    lo = (counts & np.uint64(0xFFFFFFFF)).astype(np.uint32)
    x0, x1 = _threefry2x32(0, 42, hi, lo)
    bits = x0 ^ x1
    fb = (bits >> np.uint32(9)) | np.uint32(0x3F800000)
    u = (fb.view(np.float32) - np.float32(1.0)).reshape(_NROW, _NCOL)
    idx = np.argsort(u, axis=1, kind="stable")[:, :_N].astype(np.int32)
    pad = np.repeat(idx[:, -1:], _NIDX - _N, axis=1)
    idx = np.concatenate([idx, pad], axis=1)
    row_local = (np.arange(_NROW, dtype=np.int32) % _B)[:, None]
    return (idx + row_local * _NCOL).reshape(-1)


def _sc_body(x_hbm, idx_hbm, out_hbm, xbufs, ibufs, isems, osems):
    wid = lax.axis_index("s") * _NC + lax.axis_index("c")
    xbase = wid * (_ROWS_PER_W * _NCOL)
    ibase = wid * (_ROWS_PER_W * _NIDX)
    zeros = jnp.zeros((16,), jnp.float32)

    def in_descs(b, s):
        xoff = xbase + b * (_B * _NCOL)
        ioff = ibase + b * (_B * _NIDX)
        return (
            pltpu.make_async_copy(
                x_hbm.at[pl.ds(xoff, _B * _NCOL)], xbufs[s], isems[s]
            ),
            pltpu.make_async_copy(
                idx_hbm.at[pl.ds(ioff, _B * _NIDX)], ibufs[s], isems[s]
            ),
        )

    def out_desc(b, s):
        xoff = xbase + b * (_B * _NCOL)
        return pltpu.make_async_copy(
            xbufs[s], out_hbm.at[pl.ds(xoff, _B * _NCOL)], osems[s]
        )

    def start_in(b, s):
        for d in in_descs(b, s):
            d.start()

    def wait_in(b, s):
        for d in in_descs(b, s):
            d.wait()

    def scatter(s):
        ibuf, xbuf = ibufs[s], xbufs[s]

        @pl.loop(0, _B * _NIDX // 16, unroll=8)
        def _(j):
            vidx = ibuf[pl.ds(j * 16, 16)]
            plsc.store_scatter(xbuf, [vidx], zeros)

    start_in(0, 0)
    start_in(1, 1)

    @pl.loop(0, _NBATCH, step=_NBUF)
    def _(g):
        for s in range(_NBUF):
            b = g + s
            wait_in(b, s)
            s2 = (s + 2) % _NBUF

            @pl.when(b + 2 < _NBATCH)
            def _():
                @pl.when(b >= 2)
                def _():
                    out_desc(b - 2, s2).wait()

                start_in(b + 2, s2)

            scatter(s)
            out_desc(b, s).start()

    for s in range(_NBUF):
        b = _NBATCH - _NBUF + s
        out_desc(b, b % _NBUF).wait()


@functools.partial(
    pl.kernel,
    mesh=plsc.VectorSubcoreMesh(core_axis_name="c", subcore_axis_name="s"),
    out_type=jax.ShapeDtypeStruct((_NROW * _NCOL,), jnp.float32),
    scratch_types=[
        *[pltpu.VMEM((_B * _NCOL,), jnp.float32) for _ in range(_NBUF)],
        *[pltpu.VMEM((_B * _NIDX,), jnp.int32) for _ in range(_NBUF)],
        *[pltpu.SemaphoreType.DMA for _ in range(2 * _NBUF)],
    ],
    compiler_params=pltpu.CompilerParams(needs_layout_passes=False),
)
def _sc_mask(x_hbm, idx_hbm, out_hbm, *bufs):
    _sc_body(
        x_hbm,
        idx_hbm,
        out_hbm,
        bufs[:_NBUF],
        bufs[_NBUF : 2 * _NBUF],
        bufs[2 * _NBUF : 3 * _NBUF],
        bufs[3 * _NBUF :],
    )


def kernel(x):
    idx = _idx_flat()
    out = _sc_mask(x.reshape(-1), idx)
    return out.reshape(_NROW, _NCOL)


# speedup vs baseline: 3.0515x; 2.3357x over previous
import functools

import jax
import jax.numpy as jnp
import numpy as np
from jax import lax
from jax.experimental import pallas as pl
from jax.experimental.pallas import tpu as pltpu
from jax.experimental.pallas import tpu_sc as plsc

_NROW, _NCOL = 8192, 2048
_FRACTION = 0.2
_N = int(_NCOL * _FRACTION)
_NIDX = 416

_NC, _NS = 2, 16
_NW = _NC * _NS
_ROWS_PER_W = _NROW // _NW
_B = 8
_NBATCH = _ROWS_PER_W // _B
_NBUF = 4
_COL_BITS = 11


def _threefry2x32(k0, k1, x0, x1):
    def rol(x, d):
        return ((x << np.uint32(d)) | (x >> np.uint32(32 - d))).astype(np.uint32)

    ks0, ks1 = np.uint32(k0), np.uint32(k1)
    ks2 = np.uint32(ks0 ^ ks1 ^ np.uint32(0x1BD11BDA))
    x0 = (x0 + ks0).astype(np.uint32)
    x1 = (x1 + ks1).astype(np.uint32)
    rots = ((13, 15, 26, 6), (17, 29, 16, 24))
    inject = ((ks1, ks2), (ks2, ks0), (ks0, ks1), (ks1, ks2), (ks2, ks0))
    for i in range(5):
        for d in rots[i % 2]:
            x0 = (x0 + x1).astype(np.uint32)
            x1 = rol(x1, d)
            x1 = (x1 ^ x0).astype(np.uint32)
        a, b = inject[i]
        x0 = (x0 + a).astype(np.uint32)
        x1 = (x1 + b + np.uint32(i + 1)).astype(np.uint32)
    return x0, x1


@functools.lru_cache(maxsize=1)
def _idx_flat() -> np.ndarray:
    counts = np.arange(_NROW * _NCOL, dtype=np.uint64)
    hi = (counts >> np.uint64(32)).astype(np.uint32)
    lo = (counts & np.uint64(0xFFFFFFFF)).astype(np.uint32)
    x0, x1 = _threefry2x32(0, 42, hi, lo)
    bits = x0 ^ x1
    fb = (bits >> np.uint32(9)) | np.uint32(0x3F800000)
    u = (fb.view(np.float32) - np.float32(1.0)).reshape(_NROW, _NCOL)
    idx = np.argsort(u, axis=1, kind="stable")[:, :_N].astype(np.int32)
    pad = np.repeat(idx[:, -1:], _NIDX - _N, axis=1)
    idx = np.concatenate([idx, pad], axis=1)
    row_local = (np.arange(_NROW, dtype=np.int32) % _B)[:, None]
    return (idx + row_local * _NCOL).reshape(-1)


def _sc_body(x_hbm, idx_hbm, out_hbm, xbufs, ibufs, isems, osems):
    wid = lax.axis_index("s") * _NC + lax.axis_index("c")
    rbase = wid * _ROWS_PER_W
    ibase = wid * (_ROWS_PER_W * _NIDX)
    zeros = jnp.zeros((16,), jnp.float32)

    def in_descs(b, s):
        row0 = rbase + b * _B
        ioff = ibase + b * (_B * _NIDX)
        return (
            pltpu.make_async_copy(x_hbm.at[pl.ds(row0, _B)], xbufs[s], isems[s]),
            pltpu.make_async_copy(
                idx_hbm.at[pl.ds(ioff, _B * _NIDX)], ibufs[s], isems[s]
            ),
        )

    def out_desc(b, s):
        row0 = rbase + b * _B
        return pltpu.make_async_copy(
            xbufs[s], out_hbm.at[pl.ds(row0, _B)], osems[s]
        )

    def start_in(b, s):
        for d in in_descs(b, s):
            d.start()

    def wait_in(b, s):
        for d in in_descs(b, s):
            d.wait()

    def scatter(s):
        ibuf, xbuf = ibufs[s], xbufs[s]

        @pl.loop(0, _B * _NIDX // 16, unroll=8)
        def _(j):
            vidx = ibuf[pl.ds(j * 16, 16)]
            row = lax.shift_right_logical(vidx, _COL_BITS)
            col = lax.bitwise_and(vidx, _NCOL - 1)
            plsc.store_scatter(xbuf, [row, col], zeros)

    start_in(0, 0)
    start_in(1, 1)

    @pl.loop(0, _NBATCH, step=_NBUF)
    def _(g):
        for s in range(_NBUF):
            b = g + s
            wait_in(b, s)
            s2 = (s + 2) % _NBUF

            @pl.when(b + 2 < _NBATCH)
            def _():
                @pl.when(b >= 2)
                def _():
                    out_desc(b - 2, s2).wait()

                start_in(b + 2, s2)

            scatter(s)
            out_desc(b, s).start()

    for s in range(_NBUF):
        b = _NBATCH - _NBUF + s
        out_desc(b, b % _NBUF).wait()


@functools.partial(
    pl.kernel,
    mesh=plsc.VectorSubcoreMesh(core_axis_name="c", subcore_axis_name="s"),
    out_type=jax.ShapeDtypeStruct((_NROW, _NCOL), jnp.float32),
    scratch_types=[
        *[pltpu.VMEM((_B, _NCOL), jnp.float32) for _ in range(_NBUF)],
        *[pltpu.VMEM((_B * _NIDX,), jnp.int32) for _ in range(_NBUF)],
        *[pltpu.SemaphoreType.DMA for _ in range(2 * _NBUF)],
    ],
    compiler_params=pltpu.CompilerParams(needs_layout_passes=False),
)
def _sc_mask(x_hbm, idx_hbm, out_hbm, *bufs):
    _sc_body(
        x_hbm,
        idx_hbm,
        out_hbm,
        bufs[:_NBUF],
        bufs[_NBUF : 2 * _NBUF],
        bufs[2 * _NBUF : 3 * _NBUF],
        bufs[3 * _NBUF :],
    )


def kernel(x):
    idx = _idx_flat()
    return _sc_mask(x, idx)


# speedup vs baseline: 3.5189x; 1.1532x over previous
import functools

import jax
import jax.numpy as jnp
import numpy as np
from jax import lax
from jax.experimental import pallas as pl
from jax.experimental.pallas import tpu as pltpu
from jax.experimental.pallas import tpu_sc as plsc

_NROW, _NCOL = 8192, 2048
_FRACTION = 0.2
_N = int(_NCOL * _FRACTION)
_NIDX = 416
_NPK = _NIDX // 2

_NC, _NS = 2, 16
_NW = _NC * _NS
_ROWS_PER_W = _NROW // _NW
_B = 16
_NBATCH = _ROWS_PER_W // _B
_NBUF = 3
_COL_BITS = 11


def _threefry2x32(k0, k1, x0, x1):
    def rol(x, d):
        return ((x << np.uint32(d)) | (x >> np.uint32(32 - d))).astype(np.uint32)

    ks0, ks1 = np.uint32(k0), np.uint32(k1)
    ks2 = np.uint32(ks0 ^ ks1 ^ np.uint32(0x1BD11BDA))
    x0 = (x0 + ks0).astype(np.uint32)
    x1 = (x1 + ks1).astype(np.uint32)
    rots = ((13, 15, 26, 6), (17, 29, 16, 24))
    inject = ((ks1, ks2), (ks2, ks0), (ks0, ks1), (ks1, ks2), (ks2, ks0))
    for i in range(5):
        for d in rots[i % 2]:
            x0 = (x0 + x1).astype(np.uint32)
            x1 = rol(x1, d)
            x1 = (x1 ^ x0).astype(np.uint32)
        a, b = inject[i]
        x0 = (x0 + a).astype(np.uint32)
        x1 = (x1 + b + np.uint32(i + 1)).astype(np.uint32)
    return x0, x1


@functools.lru_cache(maxsize=1)
def _idx_packed() -> np.ndarray:
    counts = np.arange(_NROW * _NCOL, dtype=np.uint64)
    hi = (counts >> np.uint64(32)).astype(np.uint32)
    lo = (counts & np.uint64(0xFFFFFFFF)).astype(np.uint32)
    x0, x1 = _threefry2x32(0, 42, hi, lo)
    bits = x0 ^ x1
    fb = (bits >> np.uint32(9)) | np.uint32(0x3F800000)
    u = (fb.view(np.float32) - np.float32(1.0)).reshape(_NROW, _NCOL)
    idx = np.argsort(u, axis=1, kind="stable")[:, :_N].astype(np.int32)
    pad = np.repeat(idx[:, -1:], _NIDX - _N, axis=1)
    idx = np.concatenate([idx, pad], axis=1)
    row_local = (np.arange(_NROW, dtype=np.int32) % _B)[:, None]
    flat = (idx + row_local * _NCOL).reshape(_NROW, _NPK, 2)
    packed = flat[:, :, 0] | (flat[:, :, 1] << 16)
    return packed.reshape(-1).astype(np.int32)


def _sc_body(x_hbm, idx_hbm, out_hbm, xbufs, ibufs, isems, osems):
    wid = lax.axis_index("s") * _NC + lax.axis_index("c")
    rbase = wid * _ROWS_PER_W
    ibase = wid * (_ROWS_PER_W * _NPK)
    zeros = jnp.zeros((16,), jnp.float32)

    def in_descs(b, s):
        row0 = rbase + b * _B
        ioff = ibase + b * (_B * _NPK)
        return (
            pltpu.make_async_copy(x_hbm.at[pl.ds(row0, _B)], xbufs[s], isems[s]),
            pltpu.make_async_copy(
                idx_hbm.at[pl.ds(ioff, _B * _NPK)], ibufs[s], isems[s]
            ),
        )

    def out_desc(b, s):
        row0 = rbase + b * _B
        return pltpu.make_async_copy(
            xbufs[s], out_hbm.at[pl.ds(row0, _B)], osems[s]
        )

    def start_in(b, s):
        for d in in_descs(b, s):
            d.start()

    def wait_in(b, s):
        for d in in_descs(b, s):
            d.wait()

    def scatter(s):
        ibuf, xbuf = ibufs[s], xbufs[s]

        @pl.loop(0, _B * _NPK // 16, unroll=8)
        def _(j):
            v = ibuf[pl.ds(j * 16, 16)]
            for part in (
                lax.bitwise_and(v, 0xFFFF),
                lax.shift_right_logical(v, 16),
            ):
                row = lax.shift_right_logical(part, _COL_BITS)
                col = lax.bitwise_and(part, _NCOL - 1)
                plsc.store_scatter(xbuf, [row, col], zeros)

    def step(b, s):
        wait_in(b, s)
        s1 = (s + 1) % _NBUF

        @pl.when(b >= 2)
        def _():
            out_desc(b - 2, s1).wait()

        start_in(b + 1, s1)
        scatter(s)
        out_desc(b, s).start()

    start_in(0, 0)

    @pl.loop(0, _NBATCH - 1, step=_NBUF)
    def _(g):
        for s in range(_NBUF):
            b = g + s

            @pl.when(b < _NBATCH - 1)
            def _():
                step(b, s)

    bt = _NBATCH - 1
    st = bt % _NBUF
    wait_in(bt, st)
    scatter(st)
    out_desc(bt, st).start()

    for b in range(_NBATCH - _NBUF, _NBATCH):
        out_desc(b, b % _NBUF).wait()


@functools.partial(
    pl.kernel,
    mesh=plsc.VectorSubcoreMesh(core_axis_name="c", subcore_axis_name="s"),
    out_type=jax.ShapeDtypeStruct((_NROW, _NCOL), jnp.float32),
    scratch_types=[
        *[pltpu.VMEM((_B, _NCOL), jnp.float32) for _ in range(_NBUF)],
        *[pltpu.VMEM((_B * _NPK,), jnp.int32) for _ in range(_NBUF)],
        *[pltpu.SemaphoreType.DMA for _ in range(2 * _NBUF)],
    ],
    compiler_params=pltpu.CompilerParams(needs_layout_passes=False),
)
def _sc_mask(x_hbm, idx_hbm, out_hbm, *bufs):
    _sc_body(
        x_hbm,
        idx_hbm,
        out_hbm,
        bufs[:_NBUF],
        bufs[_NBUF : 2 * _NBUF],
        bufs[2 * _NBUF : 3 * _NBUF],
        bufs[3 * _NBUF :],
    )


def kernel(x):
    idx = _idx_packed()
    return _sc_mask(x, idx)
